# baseline (speedup 1.0000x reference)
import functools

import jax
import jax.numpy as jnp
from jax import lax
from jax.experimental import pallas as pl
from jax.experimental.pallas import tpu as pltpu
from jax.experimental.pallas import tpu_sc as plsc

N = 10000
D = 128
NC = 2
NS = 16
NW = NC * NS
CHUNK = 128
CPW = 80
G = 8
NG0 = 16
NG1 = 4
NGT = NS * (NG0 + NG1)
E_PAD = NGT * G * CHUNK
ACC = 10112
RPT = ACC // NS


def _deg_body(dst3_hbm, ones_hbm, zeros_hbm, out_hbm, dst_v, ones_v, dacc, sem):
    cid = lax.axis_index("c")
    sid = lax.axis_index("s")
    wid = sid * NC + cid
    rows = pl.ds(sid * RPT, RPT)

    pltpu.sync_copy(ones_hbm, ones_v)
    pltpu.sync_copy(zeros_hbm.at[rows], dacc.at[rows])
    pltpu.sync_copy(dst3_hbm.at[wid], dst_v)
    plsc.subcore_barrier()

    @pl.loop(0, CPW)
    def _(j):
        pltpu.sync_copy(ones_v, dacc.at[dst_v.at[j]], add=True)

    plsc.subcore_barrier()
    pltpu.sync_copy(dacc.at[rows], out_hbm.at[cid, rows])


def _agg_body(hs_hbm, srcg_hbm, dstg_hbm, zeros_hbm, out_hbm,
              sg0, sg1, dg0, dg1, ga, gb, acc,
              sem_a, sem_b, sem_s0, sem_s1, sem_d0, sem_d1):
    cid = lax.axis_index("c")
    sid = lax.axis_index("s")
    rows = pl.ds(sid * RPT, RPT)

    ngw = jnp.where(cid == 0, NG0, NG1)
    base_g = jnp.where(cid == 0, sid * NG0, NS * NG0 + sid * NG1)

    @pl.when(cid == 0)
    def _():
        pltpu.sync_copy(hs_hbm.at[rows], acc.at[rows])

    @pl.when(cid == 1)
    def _():
        pltpu.sync_copy(zeros_hbm.at[rows], acc.at[rows])

    pltpu.sync_copy(srcg_hbm.at[base_g], sg0)
    pltpu.sync_copy(dstg_hbm.at[base_g], dg0)
    plsc.subcore_barrier()

    def wait_gather(buf, sem):
        pltpu.make_async_copy(hs_hbm.at[pl.ds(0, CHUNK)], buf, sem).wait()

    def wait_idx(sg, sem):
        pltpu.make_async_copy(srcg_hbm.at[0], sg, sem).wait()

    pltpu.async_copy(hs_hbm.at[sg0.at[0]], ga, sem_a)
    pltpu.async_copy(hs_hbm.at[sg0.at[1]], gb, sem_b)
    pltpu.async_copy(srcg_hbm.at[base_g + 1], sg1, sem_s1)
    pltpu.async_copy(dstg_hbm.at[base_g + 1], dg1, sem_d1)

    @pl.loop(0, ngw, step=2)
    def _(g):
        for half in range(2):
            s_cur, d_cur = ((sg0, dg0), (sg1, dg1))[half]
            s_nxt, d_nxt = ((sg1, dg1), (sg0, dg0))[half]
            sem_scur, sem_dcur = ((sem_s0, sem_d0), (sem_s1, sem_d1))[half]
            sem_snxt, sem_dnxt = ((sem_s1, sem_d1), (sem_s0, sem_d0))[half]
            gg = g + half
            for k in range(G):
                buf, semg = ((ga, sem_a), (gb, sem_b))[k % 2]
                wait_gather(buf, semg)
                pltpu.sync_copy(buf, acc.at[d_cur.at[k]], add=True)
                if k < G - 2:
                    pltpu.async_copy(hs_hbm.at[s_cur.at[k + 2]], buf, semg)
                elif k == G - 2:
                    @pl.when(gg + 1 < ngw)
                    def _():
                        wait_idx(s_nxt, sem_snxt)
                        wait_idx(d_nxt, sem_dnxt)
                        pltpu.async_copy(hs_hbm.at[s_nxt.at[0]], buf, semg)
                else:
                    @pl.when(gg + 1 < ngw)
                    def _():
                        pltpu.async_copy(hs_hbm.at[s_nxt.at[1]], buf, semg)

                    @pl.when(gg + 2 < ngw)
                    def _():
                        pltpu.async_copy(srcg_hbm.at[base_g + gg + 2], s_cur, sem_scur)
                        pltpu.async_copy(dstg_hbm.at[base_g + gg + 2], d_cur, sem_dcur)

    plsc.subcore_barrier()
    pltpu.sync_copy(acc.at[rows], out_hbm.at[cid, rows])


@functools.cache
def _sc_kernels():
    mesh = plsc.VectorSubcoreMesh(
        core_axis_name="c", subcore_axis_name="s", num_cores=NC, num_subcores=NS
    )
    deg = pl.kernel(
        _deg_body,
        out_type=jax.ShapeDtypeStruct((NC, ACC, D), jnp.float32),
        mesh=mesh,
        scratch_types=[
            pltpu.VMEM((CPW, CHUNK), jnp.int32),
            pltpu.VMEM((CHUNK, D), jnp.float32),
            pltpu.VMEM_SHARED((ACC, D), jnp.float32),
            pltpu.SemaphoreType.DMA,
        ],
    )
    agg = pl.kernel(
        _agg_body,
        out_type=jax.ShapeDtypeStruct((NC, ACC, D), jnp.float32),
        mesh=mesh,
        scratch_types=[
            pltpu.VMEM((G, CHUNK), jnp.int32),
            pltpu.VMEM((G, CHUNK), jnp.int32),
            pltpu.VMEM((G, CHUNK), jnp.int32),
            pltpu.VMEM((G, CHUNK), jnp.int32),
            pltpu.VMEM((CHUNK, D), jnp.float32),
            pltpu.VMEM((CHUNK, D), jnp.float32),
            pltpu.VMEM_SHARED((ACC, D), jnp.float32),
            pltpu.SemaphoreType.DMA,
            pltpu.SemaphoreType.DMA,
            pltpu.SemaphoreType.DMA,
            pltpu.SemaphoreType.DMA,
            pltpu.SemaphoreType.DMA,
            pltpu.SemaphoreType.DMA,
        ],
    )
    return deg, agg


R_TC = 512


def _mm_scale_body(x_ref, w_ref, dp_ref, o_ref, dis_ref):
    h = jnp.dot(x_ref[...], w_ref[...], preferred_element_type=jnp.float32)
    deg = dp_ref[0, :, 0] + dp_ref[1, :, 0] + 1.0
    dis = lax.rsqrt(deg)
    o_ref[...] = h * dis[:, None]
    dis_ref[...] = jnp.broadcast_to(dis[:, None], (dis.shape[0], 16))


def _combine_body(p_ref, dis_ref, b_ref, w_ref, o_ref):
    dis = dis_ref[:, 0]
    s = p_ref[0] + p_ref[1]
    z = jnp.maximum(s * dis[:, None] + b_ref[...], 0.0)
    h2 = jnp.dot(z, w_ref[...], preferred_element_type=jnp.float32)
    o_ref[...] = h2 * dis[:, None]


def _final_body(p_ref, dis_ref, b_ref, o_ref):
    dis = dis_ref[:, 0]
    s = p_ref[0] + p_ref[1]
    o_ref[...] = jnp.maximum(s * dis[:, None] + b_ref[...], 0.0)


def _mm_scale(x_pad, w, degp):
    r = R_TC
    grid = (ACC + r - 1) // r
    return pl.pallas_call(
        _mm_scale_body,
        grid=(grid,),
        in_specs=[
            pl.BlockSpec((r, D), lambda i: (i, 0)),
            pl.BlockSpec((D, D), lambda i: (0, 0)),
            pl.BlockSpec((NC, r, D), lambda i: (0, i, 0)),
        ],
        out_specs=[
            pl.BlockSpec((r, D), lambda i: (i, 0)),
            pl.BlockSpec((r, 16), lambda i: (i, 0)),
        ],
        out_shape=[
            jax.ShapeDtypeStruct((ACC, D), jnp.float32),
            jax.ShapeDtypeStruct((ACC, 16), jnp.float32),
        ],
    )(x_pad, w, degp)


def _combine(parts, dis16, b, w):
    r = R_TC
    grid = (ACC + r - 1) // r
    return pl.pallas_call(
        _combine_body,
        grid=(grid,),
        in_specs=[
            pl.BlockSpec((NC, r, D), lambda i: (0, i, 0)),
            pl.BlockSpec((r, 16), lambda i: (i, 0)),
            pl.BlockSpec((1, D), lambda i: (0, 0)),
            pl.BlockSpec((D, D), lambda i: (0, 0)),
        ],
        out_specs=pl.BlockSpec((r, D), lambda i: (i, 0)),
        out_shape=jax.ShapeDtypeStruct((ACC, D), jnp.float32),
    )(parts, dis16, b, w)


def _final(parts, dis16, b):
    r = 1000
    grid = N // r
    return pl.pallas_call(
        _final_body,
        grid=(grid,),
        in_specs=[
            pl.BlockSpec((NC, r, D), lambda i: (0, i, 0)),
            pl.BlockSpec((r, 16), lambda i: (i, 0)),
            pl.BlockSpec((1, D), lambda i: (0, 0)),
        ],
        out_specs=pl.BlockSpec((r, D), lambda i: (i, 0)),
        out_shape=jax.ShapeDtypeStruct((N, D), jnp.float32),
    )(parts, dis16, b)


def kernel(x, edge_index, W1, b1, W2, b2):
    src = edge_index[0].astype(jnp.int32)
    dst = edge_index[1].astype(jnp.int32)
    pad = E_PAD - src.shape[0]
    srcg = jnp.concatenate([src, jnp.zeros((pad,), jnp.int32)]).reshape(NGT, G, CHUNK)
    dst_pad = jnp.concatenate([dst, jnp.full((pad,), N, jnp.int32)])
    dstg = dst_pad.reshape(NGT, G, CHUNK)
    dst3 = dst_pad.reshape(NW, CPW, CHUNK)

    zeros_big = jnp.zeros((ACC, D), jnp.float32)
    ones_big = jnp.ones((CHUNK, D), jnp.float32)
    x_pad = jnp.concatenate([x, jnp.zeros((ACC - N, D), jnp.float32)], axis=0)

    deg_kernel, agg_kernel = _sc_kernels()
    degp = deg_kernel(dst3, ones_big, zeros_big)
    hs1, dis16 = _mm_scale(x_pad, W1, degp)
    p1 = agg_kernel(hs1, srcg, dstg, zeros_big)
    hs2 = _combine(p1, dis16, b1.reshape(1, D), W2)
    p2 = agg_kernel(hs2, srcg, dstg, zeros_big)
    return _final(p2, dis16, b2.reshape(1, D))


# speedup vs baseline: 1.0017x; 1.0017x over previous
import functools

import jax
import jax.numpy as jnp
from jax import lax
from jax.experimental import pallas as pl
from jax.experimental.pallas import tpu as pltpu
from jax.experimental.pallas import tpu_sc as plsc

N = 10000
D = 128
NC = 2
NS = 16
NW = NC * NS
CHUNK = 128
CPW = 80
G = 8
NG0 = 10
NG1 = 10
NGT = NS * (NG0 + NG1)
E_PAD = NGT * G * CHUNK
ACC = 10112
RPT = ACC // NS


def _deg_body(dst3_hbm, ones_hbm, zeros_hbm, out_hbm, dst_v, ones_v, dacc, sem):
    cid = lax.axis_index("c")
    sid = lax.axis_index("s")
    wid = sid * NC + cid
    rows = pl.ds(sid * RPT, RPT)

    pltpu.sync_copy(ones_hbm, ones_v)
    pltpu.sync_copy(zeros_hbm.at[rows], dacc.at[rows])
    pltpu.sync_copy(dst3_hbm.at[wid], dst_v)
    plsc.subcore_barrier()

    @pl.loop(0, CPW)
    def _(j):
        pltpu.sync_copy(ones_v, dacc.at[dst_v.at[j]], add=True)

    plsc.subcore_barrier()
    pltpu.sync_copy(dacc.at[rows], out_hbm.at[cid, rows])


def _agg_body(hs_hbm, srcg_hbm, dstg_hbm, zeros_hbm, out_hbm,
              sg0, sg1, dg0, dg1, ga, gb, acc,
              sem_a, sem_b, sem_s0, sem_s1, sem_d0, sem_d1):
    cid = lax.axis_index("c")
    sid = lax.axis_index("s")
    rows = pl.ds(sid * RPT, RPT)

    ngw = jnp.where(cid == 0, NG0, NG1)
    base_g = jnp.where(cid == 0, sid * NG0, NS * NG0 + sid * NG1)

    @pl.when(cid == 0)
    def _():
        pltpu.sync_copy(hs_hbm.at[rows], acc.at[rows])

    @pl.when(cid == 1)
    def _():
        pltpu.sync_copy(zeros_hbm.at[rows], acc.at[rows])

    pltpu.sync_copy(srcg_hbm.at[base_g], sg0)
    pltpu.sync_copy(dstg_hbm.at[base_g], dg0)
    plsc.subcore_barrier()

    def wait_gather(buf, sem):
        pltpu.make_async_copy(hs_hbm.at[pl.ds(0, CHUNK)], buf, sem).wait()

    def wait_idx(sg, sem):
        pltpu.make_async_copy(srcg_hbm.at[0], sg, sem).wait()

    pltpu.async_copy(hs_hbm.at[sg0.at[0]], ga, sem_a)
    pltpu.async_copy(hs_hbm.at[sg0.at[1]], gb, sem_b)
    pltpu.async_copy(srcg_hbm.at[base_g + 1], sg1, sem_s1)
    pltpu.async_copy(dstg_hbm.at[base_g + 1], dg1, sem_d1)

    @pl.loop(0, ngw, step=2)
    def _(g):
        for half in range(2):
            s_cur, d_cur = ((sg0, dg0), (sg1, dg1))[half]
            s_nxt, d_nxt = ((sg1, dg1), (sg0, dg0))[half]
            sem_scur, sem_dcur = ((sem_s0, sem_d0), (sem_s1, sem_d1))[half]
            sem_snxt, sem_dnxt = ((sem_s1, sem_d1), (sem_s0, sem_d0))[half]
            gg = g + half
            for k in range(G):
                buf, semg = ((ga, sem_a), (gb, sem_b))[k % 2]
                wait_gather(buf, semg)
                pltpu.sync_copy(buf, acc.at[d_cur.at[k]], add=True)
                if k < G - 2:
                    pltpu.async_copy(hs_hbm.at[s_cur.at[k + 2]], buf, semg)
                elif k == G - 2:
                    @pl.when(gg + 1 < ngw)
                    def _():
                        wait_idx(s_nxt, sem_snxt)
                        wait_idx(d_nxt, sem_dnxt)
                        pltpu.async_copy(hs_hbm.at[s_nxt.at[0]], buf, semg)
                else:
                    @pl.when(gg + 1 < ngw)
                    def _():
                        pltpu.async_copy(hs_hbm.at[s_nxt.at[1]], buf, semg)

                    @pl.when(gg + 2 < ngw)
                    def _():
                        pltpu.async_copy(srcg_hbm.at[base_g + gg + 2], s_cur, sem_scur)
                        pltpu.async_copy(dstg_hbm.at[base_g + gg + 2], d_cur, sem_dcur)

    plsc.subcore_barrier()
    pltpu.sync_copy(acc.at[rows], out_hbm.at[cid, rows])


@functools.cache
def _sc_kernels():
    mesh = plsc.VectorSubcoreMesh(
        core_axis_name="c", subcore_axis_name="s", num_cores=NC, num_subcores=NS
    )
    deg = pl.kernel(
        _deg_body,
        out_type=jax.ShapeDtypeStruct((NC, ACC, D), jnp.float32),
        mesh=mesh,
        scratch_types=[
            pltpu.VMEM((CPW, CHUNK), jnp.int32),
            pltpu.VMEM((CHUNK, D), jnp.float32),
            pltpu.VMEM_SHARED((ACC, D), jnp.float32),
            pltpu.SemaphoreType.DMA,
        ],
    )
    agg = pl.kernel(
        _agg_body,
        out_type=jax.ShapeDtypeStruct((NC, ACC, D), jnp.float32),
        mesh=mesh,
        scratch_types=[
            pltpu.VMEM((G, CHUNK), jnp.int32),
            pltpu.VMEM((G, CHUNK), jnp.int32),
            pltpu.VMEM((G, CHUNK), jnp.int32),
            pltpu.VMEM((G, CHUNK), jnp.int32),
            pltpu.VMEM((CHUNK, D), jnp.float32),
            pltpu.VMEM((CHUNK, D), jnp.float32),
            pltpu.VMEM_SHARED((ACC, D), jnp.float32),
            pltpu.SemaphoreType.DMA,
            pltpu.SemaphoreType.DMA,
            pltpu.SemaphoreType.DMA,
            pltpu.SemaphoreType.DMA,
            pltpu.SemaphoreType.DMA,
            pltpu.SemaphoreType.DMA,
        ],
    )
    return deg, agg


R_TC = 512


def _mm_scale_body(x_ref, w_ref, dp_ref, o_ref, dis_ref):
    h = jnp.dot(x_ref[...], w_ref[...], preferred_element_type=jnp.float32)
    deg = dp_ref[0, :, 0] + dp_ref[1, :, 0] + 1.0
    dis = lax.rsqrt(deg)
    o_ref[...] = h * dis[:, None]
    dis_ref[...] = jnp.broadcast_to(dis[:, None], (dis.shape[0], 16))


def _combine_body(p_ref, dis_ref, b_ref, w_ref, o_ref):
    dis = dis_ref[:, 0]
    s = p_ref[0] + p_ref[1]
    z = jnp.maximum(s * dis[:, None] + b_ref[...], 0.0)
    h2 = jnp.dot(z, w_ref[...], preferred_element_type=jnp.float32)
    o_ref[...] = h2 * dis[:, None]


def _final_body(p_ref, dis_ref, b_ref, o_ref):
    dis = dis_ref[:, 0]
    s = p_ref[0] + p_ref[1]
    o_ref[...] = jnp.maximum(s * dis[:, None] + b_ref[...], 0.0)


def _mm_scale(x_pad, w, degp):
    r = R_TC
    grid = (ACC + r - 1) // r
    return pl.pallas_call(
        _mm_scale_body,
        grid=(grid,),
        in_specs=[
            pl.BlockSpec((r, D), lambda i: (i, 0)),
            pl.BlockSpec((D, D), lambda i: (0, 0)),
            pl.BlockSpec((NC, r, D), lambda i: (0, i, 0)),
        ],
        out_specs=[
            pl.BlockSpec((r, D), lambda i: (i, 0)),
            pl.BlockSpec((r, 16), lambda i: (i, 0)),
        ],
        out_shape=[
            jax.ShapeDtypeStruct((ACC, D), jnp.float32),
            jax.ShapeDtypeStruct((ACC, 16), jnp.float32),
        ],
    )(x_pad, w, degp)


def _combine(parts, dis16, b, w):
    r = R_TC
    grid = (ACC + r - 1) // r
    return pl.pallas_call(
        _combine_body,
        grid=(grid,),
        in_specs=[
            pl.BlockSpec((NC, r, D), lambda i: (0, i, 0)),
            pl.BlockSpec((r, 16), lambda i: (i, 0)),
            pl.BlockSpec((1, D), lambda i: (0, 0)),
            pl.BlockSpec((D, D), lambda i: (0, 0)),
        ],
        out_specs=pl.BlockSpec((r, D), lambda i: (i, 0)),
        out_shape=jax.ShapeDtypeStruct((ACC, D), jnp.float32),
    )(parts, dis16, b, w)


def _final(parts, dis16, b):
    r = 1000
    grid = N // r
    return pl.pallas_call(
        _final_body,
        grid=(grid,),
        in_specs=[
            pl.BlockSpec((NC, r, D), lambda i: (0, i, 0)),
            pl.BlockSpec((r, 16), lambda i: (i, 0)),
            pl.BlockSpec((1, D), lambda i: (0, 0)),
        ],
        out_specs=pl.BlockSpec((r, D), lambda i: (i, 0)),
        out_shape=jax.ShapeDtypeStruct((N, D), jnp.float32),
    )(parts, dis16, b)


def kernel(x, edge_index, W1, b1, W2, b2):
    src = edge_index[0].astype(jnp.int32)
    dst = edge_index[1].astype(jnp.int32)
    pad = E_PAD - src.shape[0]
    srcg = jnp.concatenate([src, jnp.zeros((pad,), jnp.int32)]).reshape(NGT, G, CHUNK)
    trash = N + (jnp.arange(pad, dtype=jnp.int32) % (ACC - N))
    dst_pad = jnp.concatenate([dst, trash])
    dstg = dst_pad.reshape(NGT, G, CHUNK)
    dst3 = dst_pad.reshape(NW, CPW, CHUNK)

    zeros_big = jnp.zeros((ACC, D), jnp.float32)
    ones_big = jnp.ones((CHUNK, D), jnp.float32)
    x_pad = jnp.concatenate([x, jnp.zeros((ACC - N, D), jnp.float32)], axis=0)

    deg_kernel, agg_kernel = _sc_kernels()
    degp = deg_kernel(dst3, ones_big, zeros_big)
    hs1, dis16 = _mm_scale(x_pad, W1, degp)
    p1 = agg_kernel(hs1, srcg, dstg, zeros_big)
    hs2 = _combine(p1, dis16, b1.reshape(1, D), W2)
    p2 = agg_kernel(hs2, srcg, dstg, zeros_big)
    return _final(p2, dis16, b2.reshape(1, D))


# speedup vs baseline: 1.1508x; 1.1489x over previous
import functools

import jax
import jax.numpy as jnp
from jax import lax
from jax.experimental import pallas as pl
from jax.experimental.pallas import tpu as pltpu
from jax.experimental.pallas import tpu_sc as plsc

N = 10000
D = 128
NC = 2
NS = 16
NW = NC * NS
CHUNK = 128
CPW = 80
G = 8
NG = CPW // G
E_PAD = NW * CPW * CHUNK
ACC = 10112
RPT = ACC // NS


def _deg_body(dst3_hbm, ones_hbm, zeros_hbm, out_hbm, dst_v, ones_v, dacc, sem):
    cid = lax.axis_index("c")
    sid = lax.axis_index("s")
    wid = sid * NC + cid
    rows = pl.ds(sid * RPT, RPT)

    pltpu.sync_copy(ones_hbm, ones_v)
    pltpu.sync_copy(zeros_hbm.at[rows], dacc.at[rows])
    pltpu.sync_copy(dst3_hbm.at[wid], dst_v)
    plsc.subcore_barrier()

    @pl.loop(0, CPW)
    def _(j):
        pltpu.sync_copy(ones_v, dacc.at[dst_v.at[j]], add=True)

    plsc.subcore_barrier()
    pltpu.sync_copy(dacc.at[rows], out_hbm.at[cid, rows])


def _agg_body(hs_hbm, src4_hbm, dst3_hbm, zeros_hbm, out_hbm,
              dst_v, sg0, sg1, ga, gb, acc,
              sem_a, sem_b, sem_s0, sem_s1):
    cid = lax.axis_index("c")
    sid = lax.axis_index("s")
    wid = sid * NC + cid
    rows = pl.ds(sid * RPT, RPT)

    @pl.when(cid == 0)
    def _():
        pltpu.sync_copy(hs_hbm.at[rows], acc.at[rows])

    @pl.when(cid == 1)
    def _():
        pltpu.sync_copy(zeros_hbm.at[rows], acc.at[rows])

    pltpu.sync_copy(dst3_hbm.at[wid], dst_v)
    pltpu.sync_copy(src4_hbm.at[wid, 0], sg0)
    plsc.subcore_barrier()

    def wait_gather(buf, sem):
        pltpu.make_async_copy(hs_hbm.at[pl.ds(0, CHUNK)], buf, sem).wait()

    def wait_idx(sg, sem):
        pltpu.make_async_copy(src4_hbm.at[0, 0], sg, sem).wait()

    pltpu.async_copy(hs_hbm.at[sg0.at[0]], ga, sem_a)
    pltpu.async_copy(hs_hbm.at[sg0.at[1]], gb, sem_b)
    pltpu.async_copy(src4_hbm.at[wid, 1], sg1, sem_s1)

    @pl.loop(0, NG, step=2)
    def _(g):
        for half in range(2):
            s_cur = (sg0, sg1)[half]
            s_nxt = (sg1, sg0)[half]
            sem_cur = (sem_s0, sem_s1)[half]
            sem_nxt = (sem_s1, sem_s0)[half]
            gg = g + half
            for k in range(G):
                buf, semg = ((ga, sem_a), (gb, sem_b))[k % 2]
                wait_gather(buf, semg)
                pltpu.sync_copy(buf, acc.at[dst_v.at[gg * G + k]], add=True)
                if k < G - 2:
                    pltpu.async_copy(hs_hbm.at[s_cur.at[k + 2]], buf, semg)
                elif k == G - 2:
                    @pl.when(gg + 1 < NG)
                    def _():
                        wait_idx(s_nxt, sem_nxt)
                        pltpu.async_copy(hs_hbm.at[s_nxt.at[0]], buf, semg)
                else:
                    @pl.when(gg + 1 < NG)
                    def _():
                        pltpu.async_copy(hs_hbm.at[s_nxt.at[1]], buf, semg)

                    @pl.when(gg + 2 < NG)
                    def _():
                        pltpu.async_copy(src4_hbm.at[wid, gg + 2], s_cur, sem_cur)

    plsc.subcore_barrier()
    pltpu.sync_copy(acc.at[rows], out_hbm.at[cid, rows])


@functools.cache
def _sc_kernels():
    mesh = plsc.VectorSubcoreMesh(
        core_axis_name="c", subcore_axis_name="s", num_cores=NC, num_subcores=NS
    )
    deg = pl.kernel(
        _deg_body,
        out_type=jax.ShapeDtypeStruct((NC, ACC, D), jnp.float32),
        mesh=mesh,
        scratch_types=[
            pltpu.VMEM((CPW, CHUNK), jnp.int32),
            pltpu.VMEM((CHUNK, D), jnp.float32),
            pltpu.VMEM_SHARED((ACC, D), jnp.float32),
            pltpu.SemaphoreType.DMA,
        ],
    )
    agg = pl.kernel(
        _agg_body,
        out_type=jax.ShapeDtypeStruct((NC, ACC, D), jnp.float32),
        mesh=mesh,
        scratch_types=[
            pltpu.VMEM((CPW, CHUNK), jnp.int32),
            pltpu.VMEM((G, CHUNK), jnp.int32),
            pltpu.VMEM((G, CHUNK), jnp.int32),
            pltpu.VMEM((CHUNK, D), jnp.float32),
            pltpu.VMEM((CHUNK, D), jnp.float32),
            pltpu.VMEM_SHARED((ACC, D), jnp.float32),
            pltpu.SemaphoreType.DMA,
            pltpu.SemaphoreType.DMA,
            pltpu.SemaphoreType.DMA,
            pltpu.SemaphoreType.DMA,
        ],
    )
    return deg, agg


R_TC = 512


def _mm_scale_body(x_ref, w_ref, dp_ref, o_ref, dis_ref):
    h = jnp.dot(x_ref[...], w_ref[...], preferred_element_type=jnp.float32)
    deg = dp_ref[0, :, 0] + dp_ref[1, :, 0] + 1.0
    dis = lax.rsqrt(deg)
    o_ref[...] = h * dis[:, None]
    dis_ref[...] = jnp.broadcast_to(dis[:, None], (dis.shape[0], 16))


def _combine_body(p_ref, dis_ref, b_ref, w_ref, o_ref):
    dis = dis_ref[:, 0]
    s = p_ref[0] + p_ref[1]
    z = jnp.maximum(s * dis[:, None] + b_ref[...], 0.0)
    h2 = jnp.dot(z, w_ref[...], preferred_element_type=jnp.float32)
    o_ref[...] = h2 * dis[:, None]


def _final_body(p_ref, dis_ref, b_ref, o_ref):
    dis = dis_ref[:, 0]
    s = p_ref[0] + p_ref[1]
    o_ref[...] = jnp.maximum(s * dis[:, None] + b_ref[...], 0.0)


def _mm_scale(x_pad, w, degp):
    r = R_TC
    grid = (ACC + r - 1) // r
    return pl.pallas_call(
        _mm_scale_body,
        grid=(grid,),
        in_specs=[
            pl.BlockSpec((r, D), lambda i: (i, 0)),
            pl.BlockSpec((D, D), lambda i: (0, 0)),
            pl.BlockSpec((NC, r, D), lambda i: (0, i, 0)),
        ],
        out_specs=[
            pl.BlockSpec((r, D), lambda i: (i, 0)),
            pl.BlockSpec((r, 16), lambda i: (i, 0)),
        ],
        out_shape=[
            jax.ShapeDtypeStruct((ACC, D), jnp.float32),
            jax.ShapeDtypeStruct((ACC, 16), jnp.float32),
        ],
    )(x_pad, w, degp)


def _combine(parts, dis16, b, w):
    r = R_TC
    grid = (ACC + r - 1) // r
    return pl.pallas_call(
        _combine_body,
        grid=(grid,),
        in_specs=[
            pl.BlockSpec((NC, r, D), lambda i: (0, i, 0)),
            pl.BlockSpec((r, 16), lambda i: (i, 0)),
            pl.BlockSpec((1, D), lambda i: (0, 0)),
            pl.BlockSpec((D, D), lambda i: (0, 0)),
        ],
        out_specs=pl.BlockSpec((r, D), lambda i: (i, 0)),
        out_shape=jax.ShapeDtypeStruct((ACC, D), jnp.float32),
    )(parts, dis16, b, w)


def _final(parts, dis16, b):
    r = 1000
    grid = N // r
    return pl.pallas_call(
        _final_body,
        grid=(grid,),
        in_specs=[
            pl.BlockSpec((NC, r, D), lambda i: (0, i, 0)),
            pl.BlockSpec((r, 16), lambda i: (i, 0)),
            pl.BlockSpec((1, D), lambda i: (0, 0)),
        ],
        out_specs=pl.BlockSpec((r, D), lambda i: (i, 0)),
        out_shape=jax.ShapeDtypeStruct((N, D), jnp.float32),
    )(parts, dis16, b)


def kernel(x, edge_index, W1, b1, W2, b2):
    src = edge_index[0].astype(jnp.int32)
    dst = edge_index[1].astype(jnp.int32)
    pad = E_PAD - src.shape[0]
    src4 = jnp.concatenate([src, jnp.zeros((pad,), jnp.int32)]).reshape(NW, NG, G, CHUNK)
    trash = N + (jnp.arange(pad, dtype=jnp.int32) % (ACC - N))
    dst3 = jnp.concatenate([dst, trash]).reshape(NW, CPW, CHUNK)

    zeros_big = jnp.zeros((ACC, D), jnp.float32)
    ones_big = jnp.ones((CHUNK, D), jnp.float32)
    x_pad = jnp.concatenate([x, jnp.zeros((ACC - N, D), jnp.float32)], axis=0)

    deg_kernel, agg_kernel = _sc_kernels()
    degp = deg_kernel(dst3, ones_big, zeros_big)
    hs1, dis16 = _mm_scale(x_pad, W1, degp)
    p1 = agg_kernel(hs1, src4, dst3, zeros_big)
    hs2 = _combine(p1, dis16, b1.reshape(1, D), W2)
    p2 = agg_kernel(hs2, src4, dst3, zeros_big)
    return _final(p2, dis16, b2.reshape(1, D))
